# baseline (speedup 1.0000x reference)
import functools

import jax
import jax.numpy as jnp
import numpy as np
from jax.experimental import pallas as pl
from jax.experimental.pallas import tpu as pltpu

_BSZ, _H, _SEQ, _D = 16, 8, 4096, 128
_ROWS = _BSZ * _H
_SBLK = 4096
_NSB = _SEQ // _SBLK


def _dequant_matvec_kernel(s_ref, f8_ref, m_ref, l_ref, p1_ref, p2_ref,
                           o_ref):
    j = pl.program_id(1)
    x = f8_ref[0].astype(jnp.int8).astype(jnp.int32) << 21
    hv = jax.lax.bitcast_convert_type(x & ~jnp.int32(0x701FFFFF),
                                      jnp.float32)
    hf = jax.lax.bitcast_convert_type(x & ~jnp.int32(0x707FFFFF),
                                      jnp.float32)
    m = m_ref[0].astype(jnp.int32)
    l = l_ref[0].astype(jnp.int32)
    ce = ((m & 0xF0) | (l >> 4)).astype(jnp.float32)
    co = (((m & 0x0F) << 4) | (l & 0x0F)).astype(jnp.float32)
    dn = (((1,), (0,)), ((), ()))
    kw = dict(preferred_element_type=jnp.float32)
    c = (jax.lax.dot_general(ce, p1_ref[0], dn, **kw)
         + jax.lax.dot_general(co, p2_ref[0], dn, **kw))
    v = hv + hf * c
    srow = s_ref[0]
    part = jax.lax.dot_general(
        srow, v, (((1,), (0,)), ((), ())),
        preferred_element_type=jnp.float32)
    @pl.when(j == 0)
    def _():
        o_ref[0] = part
    @pl.when(j != 0)
    def _():
        o_ref[0] += part


def _scatter_mats():
    p1 = np.zeros((1, _D // 2, _D), dtype=np.float32)
    p2 = np.zeros((1, _D // 2, _D), dtype=np.float32)
    sc = float(2.0 ** -10)
    for jj in range(_D // 2):
        p1[0, jj, 2 * jj] = sc
        p2[0, jj, 2 * jj + 1] = sc
    return p1, p2


_NCH = 2
_BCH = _BSZ // _NCH
_CROWS = _BCH * _H


@functools.partial(jax.jit, static_argnames=())
def _run(s, v_first8, v_mid4, v_last4):
    p1, p2 = _scatter_mats()
    p1 = jnp.asarray(p1)
    p2 = jnp.asarray(p2)
    outs = []
    for ci in range(_NCH):
        b0 = ci * _BCH
        s2 = (s[b0:b0 + _BCH] * jnp.float32(2.0 ** 112)).reshape(
            _CROWS, 1, _SEQ)
        f8 = v_first8[b0:b0 + _BCH].reshape(_CROWS, _SEQ, _D)
        m = v_mid4[b0:b0 + _BCH].reshape(_CROWS, _SEQ, _D // 2)
        l = v_last4[b0:b0 + _BCH].reshape(_CROWS, _SEQ, _D // 2)
        out = pl.pallas_call(
            _dequant_matvec_kernel,
            grid=(_CROWS, _NSB),
            in_specs=[
                pl.BlockSpec((1, 1, _SBLK), lambda i, j: (i, 0, j)),
                pl.BlockSpec((1, _SBLK, _D), lambda i, j: (i, j, 0)),
                pl.BlockSpec((1, _SBLK, _D // 2), lambda i, j: (i, j, 0)),
                pl.BlockSpec((1, _SBLK, _D // 2), lambda i, j: (i, j, 0)),
                pl.BlockSpec((1, _D // 2, _D), lambda i, j: (0, 0, 0)),
                pl.BlockSpec((1, _D // 2, _D), lambda i, j: (0, 0, 0)),
            ],
            out_specs=pl.BlockSpec((1, 1, _D), lambda i, j: (i, 0, 0)),
            out_shape=jax.ShapeDtypeStruct((_CROWS, 1, _D), jnp.float32),
            compiler_params=pltpu.CompilerParams(
                dimension_semantics=("parallel", "arbitrary")),
        )(s2, f8, m, l, p1, p2)
        outs.append(out.reshape(_BCH, _H, 1, _D))
    return jnp.concatenate(outs, axis=0)


def kernel(s, v_first8, v_mid4, v_last4, v_exp_col_max, start_pos, seqlen):
    del v_exp_col_max, start_pos, seqlen
    return _run(s, v_first8, v_mid4, v_last4)


# speedup vs baseline: 1.4721x; 1.4721x over previous
import functools

import jax
import jax.numpy as jnp
import numpy as np
from jax.experimental import pallas as pl
from jax.experimental.pallas import tpu as pltpu

_BSZ, _H, _SEQ, _D = 16, 8, 4096, 128
_ROWS = _BSZ * _H
_SBLK = 4096
_NSB = _SEQ // _SBLK


def _dequant_matvec_kernel(s_ref, f8_ref, m_ref, l_ref, p1_ref, p2_ref,
                           o_ref):
    j = pl.program_id(1)
    x = f8_ref[0].astype(jnp.int8).astype(jnp.int32) << 21
    hv = jax.lax.bitcast_convert_type(x & ~jnp.int32(0x701FFFFF),
                                      jnp.float32)
    hf = jax.lax.bitcast_convert_type(x & ~jnp.int32(0x707FFFFF),
                                      jnp.float32)
    m = m_ref[0].astype(jnp.int32)
    l = l_ref[0].astype(jnp.int32)
    ce = ((m & 0xF0) | (l >> 4)).astype(jnp.float32)
    co = (((m & 0x0F) << 4) | (l & 0x0F)).astype(jnp.float32)
    dn = (((1,), (0,)), ((), ()))
    kw = dict(preferred_element_type=jnp.float32)
    c = (jax.lax.dot_general(ce, p1_ref[0], dn, **kw)
         + jax.lax.dot_general(co, p2_ref[0], dn, **kw))
    v = hv + hf * c
    srow = s_ref[0, 0] * jnp.float32(2.0 ** 112)
    part = jax.lax.dot_general(
        srow, v, (((1,), (0,)), ((), ())),
        preferred_element_type=jnp.float32)
    @pl.when(j == 0)
    def _():
        o_ref[0] = part
    @pl.when(j != 0)
    def _():
        o_ref[0] += part


def _scatter_mats():
    p1 = np.zeros((1, _D // 2, _D), dtype=np.float32)
    p2 = np.zeros((1, _D // 2, _D), dtype=np.float32)
    sc = float(2.0 ** -10)
    for jj in range(_D // 2):
        p1[0, jj, 2 * jj] = sc
        p2[0, jj, 2 * jj + 1] = sc
    return p1, p2


@functools.partial(jax.jit, static_argnames=())
def _run(s, v_first8, v_mid4, v_last4):
    f8 = v_first8.reshape(_ROWS, _SEQ, _D)
    m = v_mid4.reshape(_ROWS, _SEQ, _D // 2)
    l = v_last4.reshape(_ROWS, _SEQ, _D // 2)
    p1, p2 = _scatter_mats()
    out = pl.pallas_call(
        _dequant_matvec_kernel,
        grid=(_ROWS, _NSB),
        in_specs=[
            pl.BlockSpec((1, 1, 1, _SBLK),
                         lambda i, j: (i // _H, i % _H, 0, j)),
            pl.BlockSpec((1, _SBLK, _D), lambda i, j: (i, j, 0)),
            pl.BlockSpec((1, _SBLK, _D // 2), lambda i, j: (i, j, 0)),
            pl.BlockSpec((1, _SBLK, _D // 2), lambda i, j: (i, j, 0)),
            pl.BlockSpec((1, _D // 2, _D), lambda i, j: (0, 0, 0)),
            pl.BlockSpec((1, _D // 2, _D), lambda i, j: (0, 0, 0)),
        ],
        out_specs=pl.BlockSpec((1, 1, _D), lambda i, j: (i, 0, 0)),
        out_shape=jax.ShapeDtypeStruct((_ROWS, 1, _D), jnp.float32),
        compiler_params=pltpu.CompilerParams(
            dimension_semantics=("parallel", "arbitrary")),
    )(s, f8, m, l, jnp.asarray(p1), jnp.asarray(p2))
    return out.reshape(_BSZ, _H, 1, _D)


def kernel(s, v_first8, v_mid4, v_last4, v_exp_col_max, start_pos, seqlen):
    del v_exp_col_max, start_pos, seqlen
    return _run(s, v_first8, v_mid4, v_last4)


# speedup vs baseline: 1.5044x; 1.0219x over previous
import functools

import jax
import jax.numpy as jnp
import numpy as np
from jax.experimental import pallas as pl
from jax.experimental.pallas import tpu as pltpu

_BSZ, _H, _SEQ, _D = 16, 8, 4096, 128
_ROWS = _BSZ * _H
_SBLK = 4096
_NSB = _SEQ // _SBLK
_RB = 2


def _dequant_matvec_kernel(s_ref, f8_ref, m_ref, l_ref, p1_ref, p2_ref,
                           o_ref):
    j = pl.program_id(1)
    for r in range(_RB):
        x = f8_ref[r].astype(jnp.int8).astype(jnp.int32) << 21
        hv = jax.lax.bitcast_convert_type(x & ~jnp.int32(0x701FFFFF),
                                          jnp.float32)
        hf = jax.lax.bitcast_convert_type(x & ~jnp.int32(0x707FFFFF),
                                          jnp.float32)
        m = m_ref[r].astype(jnp.int32)
        l = l_ref[r].astype(jnp.int32)
        ce = ((m & 0xF0) | (l >> 4)).astype(jnp.float32)
        co = (((m & 0x0F) << 4) | (l & 0x0F)).astype(jnp.float32)
        dn = (((1,), (0,)), ((), ()))
        kw = dict(preferred_element_type=jnp.float32)
        c = (jax.lax.dot_general(ce, p1_ref[0], dn, **kw)
             + jax.lax.dot_general(co, p2_ref[0], dn, **kw))
        v = hv + hf * c
        srow = s_ref[0, r] * jnp.float32(2.0 ** 112)
        part = jax.lax.dot_general(
            srow, v, (((1,), (0,)), ((), ())),
            preferred_element_type=jnp.float32)
        @pl.when(j == 0)
        def _():
            o_ref[r] = part
        @pl.when(j != 0)
        def _():
            o_ref[r] += part


def _scatter_mats():
    p1 = np.zeros((1, _D // 2, _D), dtype=np.float32)
    p2 = np.zeros((1, _D // 2, _D), dtype=np.float32)
    sc = float(2.0 ** -10)
    for jj in range(_D // 2):
        p1[0, jj, 2 * jj] = sc
        p2[0, jj, 2 * jj + 1] = sc
    return p1, p2


@functools.partial(jax.jit, static_argnames=())
def _run(s, v_first8, v_mid4, v_last4):
    f8 = v_first8.reshape(_ROWS, _SEQ, _D)
    m = v_mid4.reshape(_ROWS, _SEQ, _D // 2)
    l = v_last4.reshape(_ROWS, _SEQ, _D // 2)
    p1, p2 = _scatter_mats()
    out = pl.pallas_call(
        _dequant_matvec_kernel,
        grid=(_ROWS // _RB, _NSB),
        in_specs=[
            pl.BlockSpec((1, _RB, 1, _SBLK),
                         lambda i, j: (i // (_H // _RB), i % (_H // _RB),
                                       0, j)),
            pl.BlockSpec((_RB, _SBLK, _D), lambda i, j: (i, j, 0)),
            pl.BlockSpec((_RB, _SBLK, _D // 2), lambda i, j: (i, j, 0)),
            pl.BlockSpec((_RB, _SBLK, _D // 2), lambda i, j: (i, j, 0)),
            pl.BlockSpec((1, _D // 2, _D), lambda i, j: (0, 0, 0)),
            pl.BlockSpec((1, _D // 2, _D), lambda i, j: (0, 0, 0)),
        ],
        out_specs=pl.BlockSpec((_RB, 1, _D), lambda i, j: (i, 0, 0)),
        out_shape=jax.ShapeDtypeStruct((_ROWS, 1, _D), jnp.float32),
        compiler_params=pltpu.CompilerParams(
            dimension_semantics=("parallel", "arbitrary")),
    )(s, f8, m, l, jnp.asarray(p1), jnp.asarray(p2))
    return out.reshape(_BSZ, _H, 1, _D)


def kernel(s, v_first8, v_mid4, v_last4, v_exp_col_max, start_pos, seqlen):
    del v_exp_col_max, start_pos, seqlen
    return _run(s, v_first8, v_mid4, v_last4)


# speedup vs baseline: 1.5217x; 1.0115x over previous
import functools

import jax
import jax.numpy as jnp
import numpy as np
from jax.experimental import pallas as pl
from jax.experimental.pallas import tpu as pltpu

_BSZ, _H, _SEQ, _D = 16, 8, 4096, 128
_ROWS = _BSZ * _H
_SBLK = 4096
_NSB = _SEQ // _SBLK
_RB = 4


def _dequant_matvec_kernel(s_ref, f8_ref, m_ref, l_ref, p1_ref, p2_ref,
                           o_ref):
    j = pl.program_id(1)
    for r in range(_RB):
        x = f8_ref[r].astype(jnp.int8).astype(jnp.int32) << 21
        hv = jax.lax.bitcast_convert_type(x & ~jnp.int32(0x701FFFFF),
                                          jnp.float32)
        hf = jax.lax.bitcast_convert_type(x & ~jnp.int32(0x707FFFFF),
                                          jnp.float32)
        m = m_ref[r].astype(jnp.int32)
        l = l_ref[r].astype(jnp.int32)
        ce = ((m & 0xF0) | (l >> 4)).astype(jnp.float32)
        co = (((m & 0x0F) << 4) | (l & 0x0F)).astype(jnp.float32)
        dn = (((1,), (0,)), ((), ()))
        kw = dict(preferred_element_type=jnp.float32)
        c = (jax.lax.dot_general(ce, p1_ref[0], dn, **kw)
             + jax.lax.dot_general(co, p2_ref[0], dn, **kw))
        v = hv + hf * c
        srow = s_ref[0, r] * jnp.float32(2.0 ** 112)
        part = jax.lax.dot_general(
            srow, v, (((1,), (0,)), ((), ())),
            preferred_element_type=jnp.float32)
        @pl.when(j == 0)
        def _():
            o_ref[r] = part
        @pl.when(j != 0)
        def _():
            o_ref[r] += part


def _scatter_mats():
    p1 = np.zeros((1, _D // 2, _D), dtype=np.float32)
    p2 = np.zeros((1, _D // 2, _D), dtype=np.float32)
    sc = float(2.0 ** -10)
    for jj in range(_D // 2):
        p1[0, jj, 2 * jj] = sc
        p2[0, jj, 2 * jj + 1] = sc
    return p1, p2


@functools.partial(jax.jit, static_argnames=())
def _run(s, v_first8, v_mid4, v_last4):
    f8 = v_first8.reshape(_ROWS, _SEQ, _D)
    m = v_mid4.reshape(_ROWS, _SEQ, _D // 2)
    l = v_last4.reshape(_ROWS, _SEQ, _D // 2)
    p1, p2 = _scatter_mats()
    out = pl.pallas_call(
        _dequant_matvec_kernel,
        grid=(_ROWS // _RB, _NSB),
        in_specs=[
            pl.BlockSpec((1, _RB, 1, _SBLK),
                         lambda i, j: (i // (_H // _RB), i % (_H // _RB),
                                       0, j)),
            pl.BlockSpec((_RB, _SBLK, _D), lambda i, j: (i, j, 0)),
            pl.BlockSpec((_RB, _SBLK, _D // 2), lambda i, j: (i, j, 0)),
            pl.BlockSpec((_RB, _SBLK, _D // 2), lambda i, j: (i, j, 0)),
            pl.BlockSpec((1, _D // 2, _D), lambda i, j: (0, 0, 0)),
            pl.BlockSpec((1, _D // 2, _D), lambda i, j: (0, 0, 0)),
        ],
        out_specs=pl.BlockSpec((_RB, 1, _D), lambda i, j: (i, 0, 0)),
        out_shape=jax.ShapeDtypeStruct((_ROWS, 1, _D), jnp.float32),
        compiler_params=pltpu.CompilerParams(
            dimension_semantics=("parallel", "arbitrary")),
    )(s, f8, m, l, jnp.asarray(p1), jnp.asarray(p2))
    return out.reshape(_BSZ, _H, 1, _D)


def kernel(s, v_first8, v_mid4, v_last4, v_exp_col_max, start_pos, seqlen):
    del v_exp_col_max, start_pos, seqlen
    return _run(s, v_first8, v_mid4, v_last4)
